# Initial kernel scaffold; baseline (speedup 1.0000x reference)
import jax
import jax.numpy as jnp
from jax.experimental import pallas as pl


def kernel(x, v, rho, particle_type, edge_index, edge_features, params):
    raise NotImplementedError("write your pallas kernel here")


# speedup vs baseline: 2.0659x; 2.0659x over previous
import functools

import jax
import jax.numpy as jnp
from jax import lax
from jax.experimental import pallas as pl
from jax.experimental.pallas import tpu as pltpu
from jax.experimental.pallas import tpu_sc as plsc

_INTERPRET = False


def _ln(z, g, b):
    m = jnp.mean(z, axis=-1, keepdims=True)
    var = jnp.mean((z - m) ** 2, axis=-1, keepdims=True)
    return (z - m) / jnp.sqrt(var + 1e-5) * g + b


def _dot(a, b):
    return jnp.dot(a.astype(jnp.bfloat16), b.astype(jnp.bfloat16),
                   preferred_element_type=jnp.float32)


_GC = 128


def _sc_gather_add(ts, td, src, dst):
    n_edges = src.shape[0]
    latent = ts.shape[1]
    info = plsc.get_sparse_core_info()
    nw = info.num_cores * info.num_subcores
    nchunks = n_edges // _GC
    base_ch, extra = divmod(nchunks, nw)
    mesh = plsc.VectorSubcoreMesh(core_axis_name="c", subcore_axis_name="s")

    @functools.partial(
        pl.kernel, mesh=mesh,
        out_type=jax.ShapeDtypeStruct((n_edges, latent), jnp.float32),
        scratch_types=[
            pltpu.VMEM((_GC,), jnp.int32),
            pltpu.VMEM((_GC,), jnp.int32),
            pltpu.VMEM((_GC, latent), jnp.float32),
            pltpu.SemaphoreType.DMA,
        ],
    )
    def k(ts_hbm, td_hbm, src_hbm, dst_hbm, out_hbm, sidx, didx, rows, sem):
        wid = lax.axis_index("s") * info.num_cores + lax.axis_index("c")
        start = wid * base_ch + jnp.minimum(wid, extra)
        count = base_ch + jnp.where(wid < extra, 1, 0)

        def body(i, _):
            base = (start + i) * _GC
            pltpu.sync_copy(src_hbm.at[pl.ds(base, _GC)], sidx)
            pltpu.sync_copy(dst_hbm.at[pl.ds(base, _GC)], didx)
            pltpu.async_copy(ts_hbm.at[sidx], rows, sem).wait()
            pltpu.async_copy(td_hbm.at[didx], rows, sem, add=True).wait()
            pltpu.sync_copy(rows, out_hbm.at[pl.ds(base, _GC)])
            return 0

        lax.fori_loop(0, count, body, 0)

    return k(ts, td, src, dst)


def _sc_segment_sum(e_upd_m, dst, n_nodes):
    n_edges = e_upd_m.shape[0]
    width = e_upd_m.shape[1]
    info = plsc.get_sparse_core_info()
    nc, ns = info.num_cores, info.num_subcores
    npairs = n_nodes // 2
    half0 = ((npairs // nc + 7) // 8) * 8
    sizes = (half0, npairs - half0)
    dump = 1024
    hpad = ((half0 + dump + ns * 8 - 1) // (ns * 8)) * (ns * 8)
    zrows = hpad // ns
    zfull, ztail = divmod(zrows, _GC)
    nchunks = n_edges // _GC
    base_ch, extra = divmod(nchunks, ns)
    ob = _GC
    nob = min(sizes) // ob
    tails = (sizes[0] - nob * ob, sizes[1] - nob * ob)
    mesh = plsc.VectorSubcoreMesh(core_axis_name="c", subcore_axis_name="s")

    @functools.partial(
        pl.kernel, mesh=mesh,
        out_type=jax.ShapeDtypeStruct((n_nodes // 2, width), jnp.float32),
        scratch_types=[
            pltpu.VMEM((_GC,), jnp.int32),
            pltpu.VMEM((_GC,), jnp.int32),
            pltpu.VMEM((_GC, width), jnp.float32),
            pltpu.VMEM_SHARED((hpad, width), jnp.float32),
            pltpu.SemaphoreType.DMA,
        ],
    )
    def k(eu_hbm, dst_hbm, out_hbm, idx, lidx, rows, acc, sem):
        c = lax.axis_index("c")
        s = lax.axis_index("s")
        pair_base = c * half0
        size = jnp.where(c == 0, sizes[0], sizes[1])

        def zbody(r, _):
            for j in range(width // 16):
                rows[r, pl.ds(j * 16, 16)] = jnp.zeros((16,), jnp.float32)
            return 0
        lax.fori_loop(0, _GC, zbody, 0)
        for kblk in range(zfull):
            pltpu.sync_copy(rows, acc.at[pl.ds(s * zrows + kblk * _GC, _GC)])
        if ztail:
            pltpu.sync_copy(rows.at[pl.ds(0, ztail)],
                            acc.at[pl.ds(s * zrows + zfull * _GC, ztail)])
        plsc.subcore_barrier()

        start = s * base_ch + jnp.minimum(s, extra)
        count = base_ch + jnp.where(s < extra, 1, 0)

        def body(i, _):
            base = (start + i) * _GC
            pltpu.sync_copy(dst_hbm.at[pl.ds(base, _GC)], idx)
            pltpu.sync_copy(eu_hbm.at[pl.ds(base, _GC)], rows)
            for j in range(_GC // 16):
                d16 = idx[pl.ds(j * 16, 16)]
                p16 = lax.shift_right_logical(d16, 1)
                l16 = p16 - pair_base
                inr = (l16 >= 0) & (l16 < size)
                spread = half0 + (d16 & (dump - 1))
                lidx[pl.ds(j * 16, 16)] = jnp.where(inr, l16, spread)
            pltpu.sync_copy(rows, acc.at[lidx], add=True)
            return 0

        lax.fori_loop(0, count, body, 0)
        plsc.subcore_barrier()

        def obody(i, _):
            b = s + i * ns
            pltpu.sync_copy(acc.at[pl.ds(b * ob, ob)], rows)
            pltpu.sync_copy(rows,
                            out_hbm.at[pl.ds(pair_base + b * ob, ob)])
            return 0

        lax.fori_loop(0, (nob - s + ns - 1) // ns, obody, 0)

        @pl.when((s == 0) & (c == 0))
        def _():
            pltpu.sync_copy(acc.at[pl.ds(nob * ob, tails[0])],
                            rows.at[pl.ds(0, tails[0])])
            pltpu.sync_copy(rows.at[pl.ds(0, tails[0])],
                            out_hbm.at[pl.ds(nob * ob, tails[0])])

        @pl.when((s == 0) & (c == 1))
        def _():
            pltpu.sync_copy(acc.at[pl.ds(nob * ob, tails[1])],
                            rows.at[pl.ds(0, tails[1])])
            pltpu.sync_copy(rows.at[pl.ds(0, tails[1])],
                            out_hbm.at[pl.ds(half0 + nob * ob, tails[1])])

    return k(e_upd_m, dst)


def _enc_node_body(x_r, v_r, rho_r, pt_r, wx, wv, wr, te9, b1, w2, b2, w3, b3,
                   lng, lnb, out_r):
    nt = te9.shape[0]
    oh = (pt_r[...] == lax.broadcasted_iota(jnp.int32, (pt_r.shape[0], nt), 1)
          ).astype(jnp.float32)
    rho_b = rho_r[...].astype(jnp.bfloat16).astype(jnp.float32)
    wr_b = wr[...].astype(jnp.bfloat16).astype(jnp.float32)
    emb = jnp.dot(oh, te9[...], preferred_element_type=jnp.float32,
                  precision=lax.Precision.HIGHEST)
    h = (_dot(x_r[...], wx[...]) + _dot(v_r[...], wv[...])
         + rho_b * wr_b + emb + b1[...])
    h = jnp.maximum(h, 0.0)
    h = jnp.maximum(_dot(h, w2[...]) + b2[...], 0.0)
    z = _dot(h, w3[...]) + b3[...]
    out_r[...] = _ln(z, lng[...], lnb[...])


def _enc_edge_body(ef_r, w1, b1, w2, b2, w3, b3, lng, lnb, out_r):
    h = jnp.maximum(_dot(ef_r[...], w1[...]) + b1[...], 0.0)
    h = jnp.maximum(_dot(h, w2[...]) + b2[...], 0.0)
    z = _dot(h, w3[...]) + b3[...]
    out_r[...] = _ln(z, lng[...], lnb[...])


def _tables_body(n_r, ws, wd, be1, ts_r, td_r):
    nb = n_r[...]
    t = _dot(nb, ws[...])
    d = _dot(nb, wd[...]) + be1[...]
    ts_r[...] = jnp.concatenate([t, t], axis=1)
    td_r[...] = jnp.concatenate([d, d], axis=1)


def _edge_step_body(e_r, g_r, d_r, w1e, w2, b2, w3, b3, lng, lnb, u_r, enew_r):
    eb = e_r[...]
    gb = g_r[...][:, :w1e.shape[1]]
    h = jnp.maximum(_dot(eb, w1e[...]) + gb, 0.0)
    h = jnp.maximum(_dot(h, w2[...]) + b2[...], 0.0)
    z = _dot(h, w3[...]) + b3[...]
    u = _ln(z, lng[...], lnb[...])
    par = (d_r[...] & 1).astype(jnp.float32)
    u_r[...] = jnp.concatenate([u * (1.0 - par), u * par], axis=1)
    enew_r[...] = eb + u


def _edge_step_last_body(e_r, g_r, d_r, w1e, w2, b2, w3, b3, lng, lnb, u_r):
    gb = g_r[...][:, :w1e.shape[1]]
    h = jnp.maximum(_dot(e_r[...], w1e[...]) + gb, 0.0)
    h = jnp.maximum(_dot(h, w2[...]) + b2[...], 0.0)
    z = _dot(h, w3[...]) + b3[...]
    u = _ln(z, lng[...], lnb[...])
    par = (d_r[...] & 1).astype(jnp.float32)
    u_r[...] = jnp.concatenate([u * (1.0 - par), u * par], axis=1)


def _node_step_body(n_r, a_r, w1n, w1a, b1, w2, b2, w3, b3, lng, lnb, out_r):
    nb = n_r[...]
    h = jnp.maximum(_dot(nb, w1n[...]) + _dot(a_r[...], w1a[...]) + b1[...], 0.0)
    h = jnp.maximum(_dot(h, w2[...]) + b2[...], 0.0)
    z = _dot(h, w3[...]) + b3[...]
    out_r[...] = nb + _ln(z, lng[...], lnb[...])


def _decode_body(n_r, w1, b1, w2, b2, w3, b3, out_r):
    h = jnp.maximum(_dot(n_r[...], w1[...]) + b1[...], 0.0)
    h = jnp.maximum(_dot(h, w2[...]) + b2[...], 0.0)
    out_r[...] = _dot(h, w3[...]) + b3[...]


def _full(arr):
    r = arr.ndim
    return pl.BlockSpec(arr.shape, lambda i, _r=r: (0,) * _r)


def _rows(arr, blk):
    r = arr.ndim
    return pl.BlockSpec((blk,) + arr.shape[1:],
                        lambda i, _r=r: (i,) + (0,) * (_r - 1))


def _call_rows(body, n_rows, blk, row_args, aux_args, out_shapes):
    grid = n_rows // blk
    in_specs = [_rows(a, blk) for a in row_args] + [_full(a) for a in aux_args]
    single = not isinstance(out_shapes, (list, tuple))
    outs = [out_shapes] if single else list(out_shapes)
    out_specs = [pl.BlockSpec((blk,) + s.shape[1:],
                              lambda i, _r=len(s.shape): (i,) + (0,) * (_r - 1))
                 for s in outs]
    res = pl.pallas_call(
        body,
        grid=(grid,),
        in_specs=in_specs,
        out_specs=out_specs[0] if single else out_specs,
        out_shape=out_shapes,
        compiler_params=pltpu.CompilerParams(
            dimension_semantics=("arbitrary",)),
        interpret=_INTERPRET,
    )(*row_args, *aux_args)
    return res


def kernel(x, v, rho, particle_type, edge_index, edge_features, params):
    n_nodes = x.shape[0]
    n_edges = edge_features.shape[0]
    f32 = jnp.float32
    src = edge_index[0]
    dst = edge_index[1]
    dst2 = dst.reshape(-1, 1)
    p = params

    bn = 5000
    be = 8000

    def r2(w):
        return w.reshape(1, -1)

    (w1, b1), (w2, b2), (w3, b3) = p['enc_node']
    lng, lnb = p['enc_node_ln']
    dim = x.shape[1]
    te9 = p['type_emb'] @ w1[2 * dim + 1:]
    nodes = _call_rows(
        _enc_node_body, n_nodes, bn,
        [x, v, rho.reshape(-1, 1), particle_type.reshape(-1, 1).astype(jnp.int32)],
        [w1[:dim], w1[dim:2 * dim], w1[2 * dim:2 * dim + 1], te9, r2(b1),
         w2, r2(b2), w3, r2(b3), r2(lng), r2(lnb)],
        jax.ShapeDtypeStruct((n_nodes, w3.shape[1]), f32))

    (w1, b1), (w2, b2), (w3, b3) = p['enc_edge']
    lng, lnb = p['enc_edge_ln']
    edges = _call_rows(
        _enc_edge_body, n_edges, be, [edge_features],
        [w1, r2(b1), w2, r2(b2), w3, r2(b3), r2(lng), r2(lnb)],
        jax.ShapeDtypeStruct((n_edges, w3.shape[1]), f32))

    latent = nodes.shape[1]
    nsteps = len(p['proc'])
    for step, pp in enumerate(p['proc']):
        (w1, b1), (w2, b2), (w3, b3) = pp['edge_mlp']
        lng, lnb = pp['edge_ln']
        w1e = w1[:latent]
        w1s = w1[latent:2 * latent]
        w1d = w1[2 * latent:]
        ts, td = _call_rows(
            _tables_body, n_nodes, bn, [nodes], [w1s, w1d, r2(b1)],
            [jax.ShapeDtypeStruct((n_nodes, 2 * latent), f32),
             jax.ShapeDtypeStruct((n_nodes, 2 * latent), f32)])
        g = _sc_gather_add(ts, td, src, dst)
        if step < nsteps - 1:
            e_upd_m, edges = _call_rows(
                _edge_step_body, n_edges, be, [edges, g, dst2],
                [w1e, w2, r2(b2), w3, r2(b3), r2(lng), r2(lnb)],
                [jax.ShapeDtypeStruct((n_edges, 2 * latent), f32),
                 jax.ShapeDtypeStruct((n_edges, latent), f32)])
        else:
            e_upd_m = _call_rows(
                _edge_step_last_body, n_edges, be, [edges, g, dst2],
                [w1e, w2, r2(b2), w3, r2(b3), r2(lng), r2(lnb)],
                jax.ShapeDtypeStruct((n_edges, 2 * latent), f32))
        agg = _sc_segment_sum(e_upd_m, dst, n_nodes).reshape(n_nodes, latent)
        (w1, b1), (w2, b2), (w3, b3) = pp['node_mlp']
        lng, lnb = pp['node_ln']
        nodes = _call_rows(
            _node_step_body, n_nodes, bn, [nodes, agg],
            [w1[:latent], w1[latent:], r2(b1), w2, r2(b2), w3, r2(b3),
             r2(lng), r2(lnb)],
            jax.ShapeDtypeStruct((n_nodes, latent), f32))

    (w1, b1), (w2, b2), (w3, b3) = p['dec']
    return _call_rows(
        _decode_body, n_nodes, bn, [nodes],
        [w1, r2(b1), w2, r2(b2), w3, r2(b3)],
        jax.ShapeDtypeStruct((n_nodes, w3.shape[1]), f32))


# speedup vs baseline: 2.4208x; 1.1718x over previous
import functools

import jax
import jax.numpy as jnp
from jax import lax
from jax.experimental import pallas as pl
from jax.experimental.pallas import tpu as pltpu
from jax.experimental.pallas import tpu_sc as plsc

_INTERPRET = False


def _ln(z, g, b):
    m = jnp.mean(z, axis=-1, keepdims=True)
    var = jnp.mean((z - m) ** 2, axis=-1, keepdims=True)
    return (z - m) / jnp.sqrt(var + 1e-5) * g + b


def _dot(a, b):
    return jnp.dot(a.astype(jnp.bfloat16), b.astype(jnp.bfloat16),
                   preferred_element_type=jnp.float32)


_GC = 128
_SGC = 64


def _sc_gather_add(ts, td, src, dst):
    n_edges = src.shape[0]
    latent = ts.shape[1]
    info = plsc.get_sparse_core_info()
    nw = info.num_cores * info.num_subcores
    nchunks = n_edges // _GC
    base_ch, extra = divmod(nchunks, nw)
    mesh = plsc.VectorSubcoreMesh(core_axis_name="c", subcore_axis_name="s")

    @functools.partial(
        pl.kernel, mesh=mesh,
        out_type=jax.ShapeDtypeStruct((n_edges, latent), jnp.float32),
        scratch_types=[
            pltpu.VMEM((_GC,), jnp.int32), pltpu.VMEM((_GC,), jnp.int32),
            pltpu.VMEM((_GC,), jnp.int32), pltpu.VMEM((_GC,), jnp.int32),
            pltpu.VMEM((_GC, latent), jnp.float32),
            pltpu.VMEM((_GC, latent), jnp.float32),
            pltpu.SemaphoreType.DMA, pltpu.SemaphoreType.DMA,
            pltpu.SemaphoreType.DMA, pltpu.SemaphoreType.DMA,
            pltpu.SemaphoreType.DMA, pltpu.SemaphoreType.DMA,
            pltpu.SemaphoreType.DMA, pltpu.SemaphoreType.DMA,
        ],
    )
    def k(ts_hbm, td_hbm, src_hbm, dst_hbm, out_hbm,
          si0, si1, di0, di1, r0, r1,
          smi0, smi1, smt0, smt1, smd0, smd1, smw0, smw1):
        wid = lax.axis_index("s") * info.num_cores + lax.axis_index("c")
        start = wid * base_ch + jnp.minimum(wid, extra)
        count = base_ch + jnp.where(wid < extra, 1, 0)
        sidx = (si0, si1)
        didx = (di0, di1)
        rows = (r0, r1)
        smi = (smi0, smi1)
        smt = (smt0, smt1)
        smd = (smd0, smd1)
        smw = (smw0, smw1)

        def issue_idx(ci, b):
            base = (start + ci) * _GC
            pltpu.async_copy(src_hbm.at[pl.ds(base, _GC)], sidx[b], smi[b])
            pltpu.async_copy(dst_hbm.at[pl.ds(base, _GC)], didx[b], smi[b])

        def wait_idx(b):
            pltpu.make_async_copy(src_hbm.at[pl.ds(0, _GC)], sidx[b],
                                  smi[b]).wait()
            pltpu.make_async_copy(dst_hbm.at[pl.ds(0, _GC)], didx[b],
                                  smi[b]).wait()

        issue_idx(0, 0)

        @pl.when(count > 1)
        def _():
            issue_idx(1, 1)
        wait_idx(0)
        pltpu.async_copy(ts_hbm.at[si0], r0, smt0)

        def pair(i2, _):
            for b in (0, 1):
                ci = 2 * i2 + b

                @pl.when(ci < count)
                def _():
                    pltpu.make_async_copy(ts_hbm.at[pl.ds(0, _GC)], rows[b],
                                          smt[b]).wait()
                    pltpu.async_copy(td_hbm.at[didx[b]], rows[b], smd[b],
                                     add=True)

                    @pl.when(ci + 1 < count)
                    def _():
                        @pl.when(ci >= 1)
                        def _():
                            pltpu.make_async_copy(
                                ts_hbm.at[pl.ds(0, _GC)], rows[1 - b],
                                smw[1 - b]).wait()
                        wait_idx(1 - b)
                        pltpu.async_copy(ts_hbm.at[sidx[1 - b]], rows[1 - b],
                                         smt[1 - b])

                    pltpu.make_async_copy(ts_hbm.at[pl.ds(0, _GC)], rows[b],
                                          smd[b]).wait()
                    pltpu.async_copy(rows[b],
                                     out_hbm.at[pl.ds((start + ci) * _GC,
                                                      _GC)], smw[b])

                    @pl.when(ci + 2 < count)
                    def _():
                        issue_idx(ci + 2, b)
            return 0

        lax.fori_loop(0, (count + 1) // 2, pair, 0)
        pltpu.make_async_copy(ts_hbm.at[pl.ds(0, _GC)], r0, smw0).wait()
        pltpu.make_async_copy(ts_hbm.at[pl.ds(0, _GC)], r1, smw1).wait()

    return k(ts, td, src, dst)


def _sc_segment_sum(e_upd_m, dst, n_nodes):
    n_edges = e_upd_m.shape[0]
    width = e_upd_m.shape[1]
    info = plsc.get_sparse_core_info()
    nc, ns = info.num_cores, info.num_subcores
    npairs = n_nodes // 2
    half0 = ((npairs // nc + 7) // 8) * 8
    sizes = (half0, npairs - half0)
    dump = 1024
    hpad = ((half0 + dump + ns * 8 - 1) // (ns * 8)) * (ns * 8)
    zrows = hpad // ns
    zfull, ztail = divmod(zrows, _SGC)
    nchunks = n_edges // _SGC
    base_ch, extra = divmod(nchunks, ns)
    ob = _SGC
    nob = min(sizes) // ob
    tails = (sizes[0] - nob * ob, sizes[1] - nob * ob)
    mesh = plsc.VectorSubcoreMesh(core_axis_name="c", subcore_axis_name="s")

    @functools.partial(
        pl.kernel, mesh=mesh,
        out_type=jax.ShapeDtypeStruct((n_nodes // 2, width), jnp.float32),
        scratch_types=[
            pltpu.VMEM((_SGC,), jnp.int32), pltpu.VMEM((_SGC,), jnp.int32),
            pltpu.VMEM((_SGC,), jnp.int32), pltpu.VMEM((_SGC,), jnp.int32),
            pltpu.VMEM((_SGC, width), jnp.float32),
            pltpu.VMEM((_SGC, width), jnp.float32),
            pltpu.VMEM_SHARED((hpad, width), jnp.float32),
            pltpu.SemaphoreType.DMA, pltpu.SemaphoreType.DMA,
            pltpu.SemaphoreType.DMA, pltpu.SemaphoreType.DMA,
            pltpu.SemaphoreType.DMA, pltpu.SemaphoreType.DMA,
        ],
    )
    def k(eu_hbm, dst_hbm, out_hbm, i0, i1, l0, l1, r0, r1, acc,
          smi0, smi1, smr0, smr1, sms0, sms1):
        c = lax.axis_index("c")
        s = lax.axis_index("s")
        pair_base = c * half0
        size = jnp.where(c == 0, sizes[0], sizes[1])
        idx = (i0, i1)
        lidx = (l0, l1)
        rows = (r0, r1)
        smi = (smi0, smi1)
        smr = (smr0, smr1)
        sms = (sms0, sms1)

        def zbody(r, _):
            for j in range(width // 16):
                r0[r, pl.ds(j * 16, 16)] = jnp.zeros((16,), jnp.float32)
            return 0
        lax.fori_loop(0, _SGC, zbody, 0)
        for kblk in range(zfull):
            pltpu.sync_copy(r0, acc.at[pl.ds(s * zrows + kblk * _SGC, _SGC)])
        if ztail:
            pltpu.sync_copy(r0.at[pl.ds(0, ztail)],
                            acc.at[pl.ds(s * zrows + zfull * _SGC, ztail)])
        plsc.subcore_barrier()

        start = s * base_ch + jnp.minimum(s, extra)
        count = base_ch + jnp.where(s < extra, 1, 0)

        def stage(ci, b):
            base = (start + ci) * _SGC
            pltpu.async_copy(dst_hbm.at[pl.ds(base, _SGC)], idx[b], smi[b])
            pltpu.async_copy(eu_hbm.at[pl.ds(base, _SGC)], rows[b], smr[b])

        stage(0, 0)

        def pair(i2, _):
            for b in (0, 1):
                ci = 2 * i2 + b

                @pl.when(ci < count)
                def _():
                    pltpu.make_async_copy(dst_hbm.at[pl.ds(0, _SGC)], idx[b],
                                          smi[b]).wait()
                    pltpu.make_async_copy(eu_hbm.at[pl.ds(0, _SGC)], rows[b],
                                          smr[b]).wait()
                    for j in range(_SGC // 16):
                        d16 = idx[b][pl.ds(j * 16, 16)]
                        p16 = lax.shift_right_logical(d16, 1)
                        l16 = p16 - pair_base
                        inr = (l16 >= 0) & (l16 < size)
                        spread = half0 + (d16 & (dump - 1))
                        lidx[b][pl.ds(j * 16, 16)] = jnp.where(inr, l16,
                                                               spread)
                    pltpu.async_copy(rows[b], acc.at[lidx[b]], sms[b],
                                     add=True)

                    @pl.when(ci + 1 < count)
                    def _():
                        @pl.when(ci >= 1)
                        def _():
                            pltpu.make_async_copy(
                                eu_hbm.at[pl.ds(0, _SGC)], rows[1 - b],
                                sms[1 - b]).wait()
                        stage(ci + 1, 1 - b)
            return 0

        lax.fori_loop(0, (count + 1) // 2, pair, 0)

        pltpu.make_async_copy(eu_hbm.at[pl.ds(0, _SGC)], r0, sms0).wait()
        pltpu.make_async_copy(eu_hbm.at[pl.ds(0, _SGC)], r1, sms1).wait()
        plsc.subcore_barrier()

        def obody(i, _):
            b = s + i * ns
            pltpu.sync_copy(acc.at[pl.ds(b * ob, ob)], r0)
            pltpu.sync_copy(r0,
                            out_hbm.at[pl.ds(pair_base + b * ob, ob)])
            return 0

        lax.fori_loop(0, (nob - s + ns - 1) // ns, obody, 0)

        @pl.when((s == 0) & (c == 0))
        def _():
            pltpu.sync_copy(acc.at[pl.ds(nob * ob, tails[0])],
                            r0.at[pl.ds(0, tails[0])])
            pltpu.sync_copy(r0.at[pl.ds(0, tails[0])],
                            out_hbm.at[pl.ds(nob * ob, tails[0])])

        @pl.when((s == 0) & (c == 1))
        def _():
            pltpu.sync_copy(acc.at[pl.ds(nob * ob, tails[1])],
                            r0.at[pl.ds(0, tails[1])])
            pltpu.sync_copy(r0.at[pl.ds(0, tails[1])],
                            out_hbm.at[pl.ds(half0 + nob * ob, tails[1])])

    return k(e_upd_m, dst)


def _enc_node_body(x_r, v_r, rho_r, pt_r, wx, wv, wr, te9, b1, w2, b2, w3, b3,
                   lng, lnb, out_r):
    nt = te9.shape[0]
    oh = (pt_r[...] == lax.broadcasted_iota(jnp.int32, (pt_r.shape[0], nt), 1)
          ).astype(jnp.float32)
    rho_b = rho_r[...].astype(jnp.bfloat16).astype(jnp.float32)
    wr_b = wr[...].astype(jnp.bfloat16).astype(jnp.float32)
    emb = jnp.dot(oh, te9[...], preferred_element_type=jnp.float32,
                  precision=lax.Precision.HIGHEST)
    h = (_dot(x_r[...], wx[...]) + _dot(v_r[...], wv[...])
         + rho_b * wr_b + emb + b1[...])
    h = jnp.maximum(h, 0.0)
    h = jnp.maximum(_dot(h, w2[...]) + b2[...], 0.0)
    z = _dot(h, w3[...]) + b3[...]
    out_r[...] = _ln(z, lng[...], lnb[...])


def _enc_edge_body(ef_r, w1, b1, w2, b2, w3, b3, lng, lnb, out_r):
    h = jnp.maximum(_dot(ef_r[...], w1[...]) + b1[...], 0.0)
    h = jnp.maximum(_dot(h, w2[...]) + b2[...], 0.0)
    z = _dot(h, w3[...]) + b3[...]
    out_r[...] = _ln(z, lng[...], lnb[...])


def _tables_body(n_r, ws, wd, be1, ts_r, td_r):
    nb = n_r[...]
    t = _dot(nb, ws[...])
    d = _dot(nb, wd[...]) + be1[...]
    ts_r[...] = jnp.concatenate([t, t], axis=1)
    td_r[...] = jnp.concatenate([d, d], axis=1)


def _edge_first_body(ef_r, g_r, d_r, ew1, eb1, ew2, eb2, ew3, eb3, elng, elnb,
                     w1e, w2, b2, w3, b3, lng, lnb, u_r, enew_r):
    h = jnp.maximum(_dot(ef_r[...], ew1[...]) + eb1[...], 0.0)
    h = jnp.maximum(_dot(h, ew2[...]) + eb2[...], 0.0)
    e0 = _ln(_dot(h, ew3[...]) + eb3[...], elng[...], elnb[...])
    gb = g_r[...][:, :w1e.shape[1]]
    h = jnp.maximum(_dot(e0, w1e[...]) + gb, 0.0)
    h = jnp.maximum(_dot(h, w2[...]) + b2[...], 0.0)
    z = _dot(h, w3[...]) + b3[...]
    u = _ln(z, lng[...], lnb[...])
    par = (d_r[...] & 1).astype(jnp.float32)
    u_r[...] = jnp.concatenate([u * (1.0 - par), u * par], axis=1)
    enew_r[...] = e0 + u


def _edge_step_body(e_r, g_r, d_r, w1e, w2, b2, w3, b3, lng, lnb, u_r, enew_r):
    eb = e_r[...]
    gb = g_r[...][:, :w1e.shape[1]]
    h = jnp.maximum(_dot(eb, w1e[...]) + gb, 0.0)
    h = jnp.maximum(_dot(h, w2[...]) + b2[...], 0.0)
    z = _dot(h, w3[...]) + b3[...]
    u = _ln(z, lng[...], lnb[...])
    par = (d_r[...] & 1).astype(jnp.float32)
    u_r[...] = jnp.concatenate([u * (1.0 - par), u * par], axis=1)
    enew_r[...] = eb + u


def _edge_step_last_body(e_r, g_r, d_r, w1e, w2, b2, w3, b3, lng, lnb, u_r):
    gb = g_r[...][:, :w1e.shape[1]]
    h = jnp.maximum(_dot(e_r[...], w1e[...]) + gb, 0.0)
    h = jnp.maximum(_dot(h, w2[...]) + b2[...], 0.0)
    z = _dot(h, w3[...]) + b3[...]
    u = _ln(z, lng[...], lnb[...])
    par = (d_r[...] & 1).astype(jnp.float32)
    u_r[...] = jnp.concatenate([u * (1.0 - par), u * par], axis=1)


def _node_step_body(n_r, a_r, w1n, w1a, b1, w2, b2, w3, b3, lng, lnb, out_r):
    nb = n_r[...]
    h = jnp.maximum(_dot(nb, w1n[...]) + _dot(a_r[...], w1a[...]) + b1[...], 0.0)
    h = jnp.maximum(_dot(h, w2[...]) + b2[...], 0.0)
    z = _dot(h, w3[...]) + b3[...]
    out_r[...] = nb + _ln(z, lng[...], lnb[...])


def _decode_body(n_r, w1, b1, w2, b2, w3, b3, out_r):
    h = jnp.maximum(_dot(n_r[...], w1[...]) + b1[...], 0.0)
    h = jnp.maximum(_dot(h, w2[...]) + b2[...], 0.0)
    out_r[...] = _dot(h, w3[...]) + b3[...]


def _full(arr):
    r = arr.ndim
    return pl.BlockSpec(arr.shape, lambda i, _r=r: (0,) * _r)


def _rows(arr, blk):
    r = arr.ndim
    return pl.BlockSpec((blk,) + arr.shape[1:],
                        lambda i, _r=r: (i,) + (0,) * (_r - 1))


def _call_rows(body, n_rows, blk, row_args, aux_args, out_shapes):
    grid = n_rows // blk
    in_specs = [_rows(a, blk) for a in row_args] + [_full(a) for a in aux_args]
    single = not isinstance(out_shapes, (list, tuple))
    outs = [out_shapes] if single else list(out_shapes)
    out_specs = [pl.BlockSpec((blk,) + s.shape[1:],
                              lambda i, _r=len(s.shape): (i,) + (0,) * (_r - 1))
                 for s in outs]
    res = pl.pallas_call(
        body,
        grid=(grid,),
        in_specs=in_specs,
        out_specs=out_specs[0] if single else out_specs,
        out_shape=out_shapes,
        compiler_params=pltpu.CompilerParams(
            dimension_semantics=("arbitrary",)),
        interpret=_INTERPRET,
    )(*row_args, *aux_args)
    return res


def kernel(x, v, rho, particle_type, edge_index, edge_features, params):
    n_nodes = x.shape[0]
    n_edges = edge_features.shape[0]
    f32 = jnp.float32
    src = edge_index[0]
    dst = edge_index[1]
    dst2 = dst.reshape(-1, 1)
    p = params

    bn = 5000
    be = 8000

    def r2(w):
        return w.reshape(1, -1)

    (w1, b1), (w2, b2), (w3, b3) = p['enc_node']
    lng, lnb = p['enc_node_ln']
    dim = x.shape[1]
    te9 = p['type_emb'] @ w1[2 * dim + 1:]
    nodes = _call_rows(
        _enc_node_body, n_nodes, bn,
        [x, v, rho.reshape(-1, 1), particle_type.reshape(-1, 1).astype(jnp.int32)],
        [w1[:dim], w1[dim:2 * dim], w1[2 * dim:2 * dim + 1], te9, r2(b1),
         w2, r2(b2), w3, r2(b3), r2(lng), r2(lnb)],
        jax.ShapeDtypeStruct((n_nodes, w3.shape[1]), f32))

    enc_e = p['enc_edge']
    enc_e_ln = p['enc_edge_ln']
    edges = None

    latent = nodes.shape[1]
    nsteps = len(p['proc'])
    for step, pp in enumerate(p['proc']):
        (w1, b1), (w2, b2), (w3, b3) = pp['edge_mlp']
        lng, lnb = pp['edge_ln']
        w1e = w1[:latent]
        w1s = w1[latent:2 * latent]
        w1d = w1[2 * latent:]
        ts, td = _call_rows(
            _tables_body, n_nodes, bn, [nodes], [w1s, w1d, r2(b1)],
            [jax.ShapeDtypeStruct((n_nodes, 2 * latent), f32),
             jax.ShapeDtypeStruct((n_nodes, 2 * latent), f32)])
        g = _sc_gather_add(ts, td, src, dst)
        if step == 0:
            (ew1, eb1), (ew2, eb2), (ew3, eb3) = enc_e
            elng, elnb = enc_e_ln
            e_upd_m, edges = _call_rows(
                _edge_first_body, n_edges, be, [edge_features, g, dst2],
                [ew1, r2(eb1), ew2, r2(eb2), ew3, r2(eb3), r2(elng), r2(elnb),
                 w1e, w2, r2(b2), w3, r2(b3), r2(lng), r2(lnb)],
                [jax.ShapeDtypeStruct((n_edges, 2 * latent), f32),
                 jax.ShapeDtypeStruct((n_edges, latent), f32)])
        elif step < nsteps - 1:
            e_upd_m, edges = _call_rows(
                _edge_step_body, n_edges, be, [edges, g, dst2],
                [w1e, w2, r2(b2), w3, r2(b3), r2(lng), r2(lnb)],
                [jax.ShapeDtypeStruct((n_edges, 2 * latent), f32),
                 jax.ShapeDtypeStruct((n_edges, latent), f32)])
        else:
            e_upd_m = _call_rows(
                _edge_step_last_body, n_edges, be, [edges, g, dst2],
                [w1e, w2, r2(b2), w3, r2(b3), r2(lng), r2(lnb)],
                jax.ShapeDtypeStruct((n_edges, 2 * latent), f32))
        agg = _sc_segment_sum(e_upd_m, dst, n_nodes).reshape(n_nodes, latent)
        (w1, b1), (w2, b2), (w3, b3) = pp['node_mlp']
        lng, lnb = pp['node_ln']
        nodes = _call_rows(
            _node_step_body, n_nodes, bn, [nodes, agg],
            [w1[:latent], w1[latent:], r2(b1), w2, r2(b2), w3, r2(b3),
             r2(lng), r2(lnb)],
            jax.ShapeDtypeStruct((n_nodes, latent), f32))

    (w1, b1), (w2, b2), (w3, b3) = p['dec']
    return _call_rows(
        _decode_body, n_nodes, bn, [nodes],
        [w1, r2(b1), w2, r2(b2), w3, r2(b3)],
        jax.ShapeDtypeStruct((n_nodes, w3.shape[1]), f32))


# speedup vs baseline: 2.4213x; 1.0002x over previous
import functools

import jax
import jax.numpy as jnp
from jax import lax
from jax.experimental import pallas as pl
from jax.experimental.pallas import tpu as pltpu
from jax.experimental.pallas import tpu_sc as plsc


def _ln(z, g, b):
    m = jnp.mean(z, axis=-1, keepdims=True)
    var = jnp.mean((z - m) ** 2, axis=-1, keepdims=True)
    return (z - m) / jnp.sqrt(var + 1e-5) * g + b


def _dot(a, b):
    return jnp.dot(a.astype(jnp.bfloat16), b.astype(jnp.bfloat16),
                   preferred_element_type=jnp.float32)


_GC = 128
_SGC = 64


def _sc_gather_add(ts, td, src, dst):
    n_edges = src.shape[0]
    latent = ts.shape[1]
    info = plsc.get_sparse_core_info()
    nw = info.num_cores * info.num_subcores
    nchunks = n_edges // _GC
    base_ch, extra = divmod(nchunks, nw)
    mesh = plsc.VectorSubcoreMesh(core_axis_name="c", subcore_axis_name="s")

    @functools.partial(
        pl.kernel, mesh=mesh,
        out_type=jax.ShapeDtypeStruct((n_edges, latent), jnp.float32),
        scratch_types=[
            pltpu.VMEM((_GC,), jnp.int32), pltpu.VMEM((_GC,), jnp.int32),
            pltpu.VMEM((_GC,), jnp.int32), pltpu.VMEM((_GC,), jnp.int32),
            pltpu.VMEM((_GC, latent), jnp.float32),
            pltpu.VMEM((_GC, latent), jnp.float32),
            pltpu.SemaphoreType.DMA, pltpu.SemaphoreType.DMA,
            pltpu.SemaphoreType.DMA, pltpu.SemaphoreType.DMA,
            pltpu.SemaphoreType.DMA, pltpu.SemaphoreType.DMA,
            pltpu.SemaphoreType.DMA, pltpu.SemaphoreType.DMA,
        ],
    )
    def k(ts_hbm, td_hbm, src_hbm, dst_hbm, out_hbm,
          si0, si1, di0, di1, r0, r1,
          smi0, smi1, smt0, smt1, smd0, smd1, smw0, smw1):
        wid = lax.axis_index("s") * info.num_cores + lax.axis_index("c")
        start = wid * base_ch + jnp.minimum(wid, extra)
        count = base_ch + jnp.where(wid < extra, 1, 0)
        sidx = (si0, si1)
        didx = (di0, di1)
        rows = (r0, r1)
        smi = (smi0, smi1)
        smt = (smt0, smt1)
        smd = (smd0, smd1)
        smw = (smw0, smw1)

        def issue_idx(ci, b):
            base = (start + ci) * _GC
            pltpu.async_copy(src_hbm.at[pl.ds(base, _GC)], sidx[b], smi[b])
            pltpu.async_copy(dst_hbm.at[pl.ds(base, _GC)], didx[b], smi[b])

        def wait_idx(b):
            pltpu.make_async_copy(src_hbm.at[pl.ds(0, _GC)], sidx[b],
                                  smi[b]).wait()
            pltpu.make_async_copy(dst_hbm.at[pl.ds(0, _GC)], didx[b],
                                  smi[b]).wait()

        issue_idx(0, 0)

        @pl.when(count > 1)
        def _():
            issue_idx(1, 1)
        wait_idx(0)
        pltpu.async_copy(ts_hbm.at[si0], r0, smt0)

        def pair(i2, _):
            for b in (0, 1):
                ci = 2 * i2 + b

                @pl.when(ci < count)
                def _():
                    pltpu.make_async_copy(ts_hbm.at[pl.ds(0, _GC)], rows[b],
                                          smt[b]).wait()
                    pltpu.async_copy(td_hbm.at[didx[b]], rows[b], smd[b],
                                     add=True)

                    @pl.when(ci + 1 < count)
                    def _():
                        @pl.when(ci >= 1)
                        def _():
                            pltpu.make_async_copy(
                                ts_hbm.at[pl.ds(0, _GC)], rows[1 - b],
                                smw[1 - b]).wait()
                        wait_idx(1 - b)
                        pltpu.async_copy(ts_hbm.at[sidx[1 - b]], rows[1 - b],
                                         smt[1 - b])

                    pltpu.make_async_copy(ts_hbm.at[pl.ds(0, _GC)], rows[b],
                                          smd[b]).wait()
                    pltpu.async_copy(rows[b],
                                     out_hbm.at[pl.ds((start + ci) * _GC,
                                                      _GC)], smw[b])

                    @pl.when(ci + 2 < count)
                    def _():
                        issue_idx(ci + 2, b)
            return 0

        lax.fori_loop(0, (count + 1) // 2, pair, 0)
        pltpu.make_async_copy(ts_hbm.at[pl.ds(0, _GC)], r0, smw0).wait()
        pltpu.make_async_copy(ts_hbm.at[pl.ds(0, _GC)], r1, smw1).wait()

    return k(ts, td, src, dst)


def _sc_segment_sum(e_upd_m, dst, n_nodes):
    n_edges = e_upd_m.shape[0]
    width = e_upd_m.shape[1]
    info = plsc.get_sparse_core_info()
    nc, ns = info.num_cores, info.num_subcores
    npairs = n_nodes // 2
    half0 = ((npairs // nc + 7) // 8) * 8
    sizes = (half0, npairs - half0)
    dump = 1024
    hpad = ((half0 + dump + ns * 8 - 1) // (ns * 8)) * (ns * 8)
    zrows = hpad // ns
    zfull, ztail = divmod(zrows, _SGC)
    nchunks = n_edges // _SGC
    base_ch, extra = divmod(nchunks, ns)
    ob = _SGC
    nob = min(sizes) // ob
    tails = (sizes[0] - nob * ob, sizes[1] - nob * ob)
    mesh = plsc.VectorSubcoreMesh(core_axis_name="c", subcore_axis_name="s")

    @functools.partial(
        pl.kernel, mesh=mesh,
        out_type=jax.ShapeDtypeStruct((n_nodes // 2, width), jnp.float32),
        scratch_types=[
            pltpu.VMEM((_SGC,), jnp.int32), pltpu.VMEM((_SGC,), jnp.int32),
            pltpu.VMEM((_SGC,), jnp.int32), pltpu.VMEM((_SGC,), jnp.int32),
            pltpu.VMEM((_SGC, width), jnp.float32),
            pltpu.VMEM((_SGC, width), jnp.float32),
            pltpu.VMEM_SHARED((hpad, width), jnp.float32),
            pltpu.SemaphoreType.DMA, pltpu.SemaphoreType.DMA,
            pltpu.SemaphoreType.DMA, pltpu.SemaphoreType.DMA,
            pltpu.SemaphoreType.DMA, pltpu.SemaphoreType.DMA,
        ],
    )
    def k(eu_hbm, dst_hbm, out_hbm, i0, i1, l0, l1, r0, r1, acc,
          smi0, smi1, smr0, smr1, sms0, sms1):
        c = lax.axis_index("c")
        s = lax.axis_index("s")
        pair_base = c * half0
        size = jnp.where(c == 0, sizes[0], sizes[1])
        idx = (i0, i1)
        lidx = (l0, l1)
        rows = (r0, r1)
        smi = (smi0, smi1)
        smr = (smr0, smr1)
        sms = (sms0, sms1)

        def zbody(r, _):
            for j in range(width // 16):
                r0[r, pl.ds(j * 16, 16)] = jnp.zeros((16,), jnp.float32)
            return 0
        lax.fori_loop(0, _SGC, zbody, 0)
        for kblk in range(zfull):
            pltpu.sync_copy(r0, acc.at[pl.ds(s * zrows + kblk * _SGC, _SGC)])
        if ztail:
            pltpu.sync_copy(r0.at[pl.ds(0, ztail)],
                            acc.at[pl.ds(s * zrows + zfull * _SGC, ztail)])
        plsc.subcore_barrier()

        start = s * base_ch + jnp.minimum(s, extra)
        count = base_ch + jnp.where(s < extra, 1, 0)

        def stage(ci, b):
            base = (start + ci) * _SGC
            pltpu.async_copy(dst_hbm.at[pl.ds(base, _SGC)], idx[b], smi[b])
            pltpu.async_copy(eu_hbm.at[pl.ds(base, _SGC)], rows[b], smr[b])

        stage(0, 0)

        def pair(i2, _):
            for b in (0, 1):
                ci = 2 * i2 + b

                @pl.when(ci < count)
                def _():
                    pltpu.make_async_copy(dst_hbm.at[pl.ds(0, _SGC)], idx[b],
                                          smi[b]).wait()
                    pltpu.make_async_copy(eu_hbm.at[pl.ds(0, _SGC)], rows[b],
                                          smr[b]).wait()
                    for j in range(_SGC // 16):
                        d16 = idx[b][pl.ds(j * 16, 16)]
                        p16 = lax.shift_right_logical(d16, 1)
                        l16 = p16 - pair_base
                        inr = (l16 >= 0) & (l16 < size)
                        spread = half0 + (d16 & (dump - 1))
                        lidx[b][pl.ds(j * 16, 16)] = jnp.where(inr, l16,
                                                               spread)
                    pltpu.async_copy(rows[b], acc.at[lidx[b]], sms[b],
                                     add=True)

                    @pl.when(ci + 1 < count)
                    def _():
                        @pl.when(ci >= 1)
                        def _():
                            pltpu.make_async_copy(
                                eu_hbm.at[pl.ds(0, _SGC)], rows[1 - b],
                                sms[1 - b]).wait()
                        stage(ci + 1, 1 - b)
            return 0

        lax.fori_loop(0, (count + 1) // 2, pair, 0)

        pltpu.make_async_copy(eu_hbm.at[pl.ds(0, _SGC)], r0, sms0).wait()
        pltpu.make_async_copy(eu_hbm.at[pl.ds(0, _SGC)], r1, sms1).wait()
        plsc.subcore_barrier()

        def obody(i, _):
            b = s + i * ns
            pltpu.sync_copy(acc.at[pl.ds(b * ob, ob)], r0)
            pltpu.sync_copy(r0,
                            out_hbm.at[pl.ds(pair_base + b * ob, ob)])
            return 0

        lax.fori_loop(0, (nob - s + ns - 1) // ns, obody, 0)

        @pl.when((s == 0) & (c == 0))
        def _():
            pltpu.sync_copy(acc.at[pl.ds(nob * ob, tails[0])],
                            r0.at[pl.ds(0, tails[0])])
            pltpu.sync_copy(r0.at[pl.ds(0, tails[0])],
                            out_hbm.at[pl.ds(nob * ob, tails[0])])

        @pl.when((s == 0) & (c == 1))
        def _():
            pltpu.sync_copy(acc.at[pl.ds(nob * ob, tails[1])],
                            r0.at[pl.ds(0, tails[1])])
            pltpu.sync_copy(r0.at[pl.ds(0, tails[1])],
                            out_hbm.at[pl.ds(half0 + nob * ob, tails[1])])

    return k(e_upd_m, dst)


def _enc_node_body(x_r, v_r, rho_r, pt_r, wx, wv, wr, te9, b1, w2, b2, w3, b3,
                   lng, lnb, out_r):
    nt = te9.shape[0]
    oh = (pt_r[...] == lax.broadcasted_iota(jnp.int32, (pt_r.shape[0], nt), 1)
          ).astype(jnp.float32)
    rho_b = rho_r[...].astype(jnp.bfloat16).astype(jnp.float32)
    wr_b = wr[...].astype(jnp.bfloat16).astype(jnp.float32)
    emb = jnp.dot(oh, te9[...], preferred_element_type=jnp.float32,
                  precision=lax.Precision.HIGHEST)
    h = (_dot(x_r[...], wx[...]) + _dot(v_r[...], wv[...])
         + rho_b * wr_b + emb + b1[...])
    h = jnp.maximum(h, 0.0)
    h = jnp.maximum(_dot(h, w2[...]) + b2[...], 0.0)
    z = _dot(h, w3[...]) + b3[...]
    out_r[...] = _ln(z, lng[...], lnb[...])


def _tables_body(n_r, ws, wd, be1, ts_r, td_r):
    nb = n_r[...]
    t = _dot(nb, ws[...])
    d = _dot(nb, wd[...]) + be1[...]
    ts_r[...] = jnp.concatenate([t, t], axis=1)
    td_r[...] = jnp.concatenate([d, d], axis=1)


def _edge_first_body(ef_r, g_r, d_r, ew1, eb1, ew2, eb2, ew3, eb3, elng, elnb,
                     w1e, w2, b2, w3, b3, lng, lnb, u_r, enew_r):
    h = jnp.maximum(_dot(ef_r[...], ew1[...]) + eb1[...], 0.0)
    h = jnp.maximum(_dot(h, ew2[...]) + eb2[...], 0.0)
    e0 = _ln(_dot(h, ew3[...]) + eb3[...], elng[...], elnb[...])
    gb = g_r[...][:, :w1e.shape[1]]
    h = jnp.maximum(_dot(e0, w1e[...]) + gb, 0.0)
    h = jnp.maximum(_dot(h, w2[...]) + b2[...], 0.0)
    z = _dot(h, w3[...]) + b3[...]
    u = _ln(z, lng[...], lnb[...])
    par = (d_r[...] & 1).astype(jnp.float32)
    u_r[...] = jnp.concatenate([u * (1.0 - par), u * par], axis=1)
    enew_r[...] = e0 + u


def _edge_step_body(e_r, g_r, d_r, w1e, w2, b2, w3, b3, lng, lnb, u_r, enew_r):
    eb = e_r[...]
    gb = g_r[...][:, :w1e.shape[1]]
    h = jnp.maximum(_dot(eb, w1e[...]) + gb, 0.0)
    h = jnp.maximum(_dot(h, w2[...]) + b2[...], 0.0)
    z = _dot(h, w3[...]) + b3[...]
    u = _ln(z, lng[...], lnb[...])
    par = (d_r[...] & 1).astype(jnp.float32)
    u_r[...] = jnp.concatenate([u * (1.0 - par), u * par], axis=1)
    enew_r[...] = eb + u


def _edge_step_last_body(e_r, g_r, d_r, w1e, w2, b2, w3, b3, lng, lnb, u_r):
    gb = g_r[...][:, :w1e.shape[1]]
    h = jnp.maximum(_dot(e_r[...], w1e[...]) + gb, 0.0)
    h = jnp.maximum(_dot(h, w2[...]) + b2[...], 0.0)
    z = _dot(h, w3[...]) + b3[...]
    u = _ln(z, lng[...], lnb[...])
    par = (d_r[...] & 1).astype(jnp.float32)
    u_r[...] = jnp.concatenate([u * (1.0 - par), u * par], axis=1)


def _node_step_body(n_r, a_r, w1n, w1a, b1, w2, b2, w3, b3, lng, lnb, out_r):
    nb = n_r[...]
    h = jnp.maximum(_dot(nb, w1n[...]) + _dot(a_r[...], w1a[...]) + b1[...], 0.0)
    h = jnp.maximum(_dot(h, w2[...]) + b2[...], 0.0)
    z = _dot(h, w3[...]) + b3[...]
    out_r[...] = nb + _ln(z, lng[...], lnb[...])


def _decode_body(n_r, w1, b1, w2, b2, w3, b3, out_r):
    h = jnp.maximum(_dot(n_r[...], w1[...]) + b1[...], 0.0)
    h = jnp.maximum(_dot(h, w2[...]) + b2[...], 0.0)
    out_r[...] = _dot(h, w3[...]) + b3[...]


def _full(arr):
    r = arr.ndim
    return pl.BlockSpec(arr.shape, lambda i, _r=r: (0,) * _r)


def _rows(arr, blk):
    r = arr.ndim
    return pl.BlockSpec((blk,) + arr.shape[1:],
                        lambda i, _r=r: (i,) + (0,) * (_r - 1))


def _call_rows(body, n_rows, blk, row_args, aux_args, out_shapes):
    grid = n_rows // blk
    in_specs = [_rows(a, blk) for a in row_args] + [_full(a) for a in aux_args]
    single = not isinstance(out_shapes, (list, tuple))
    outs = [out_shapes] if single else list(out_shapes)
    out_specs = [pl.BlockSpec((blk,) + s.shape[1:],
                              lambda i, _r=len(s.shape): (i,) + (0,) * (_r - 1))
                 for s in outs]
    res = pl.pallas_call(
        body,
        grid=(grid,),
        in_specs=in_specs,
        out_specs=out_specs[0] if single else out_specs,
        out_shape=out_shapes,
        compiler_params=pltpu.CompilerParams(
            dimension_semantics=("arbitrary",)),
    )(*row_args, *aux_args)
    return res


def kernel(x, v, rho, particle_type, edge_index, edge_features, params):
    n_nodes = x.shape[0]
    n_edges = edge_features.shape[0]
    f32 = jnp.float32
    src = edge_index[0]
    dst = edge_index[1]
    dst2 = dst.reshape(-1, 1)
    p = params

    bn = 5000
    be = 8000

    def r2(w):
        return w.reshape(1, -1)

    (w1, b1), (w2, b2), (w3, b3) = p['enc_node']
    lng, lnb = p['enc_node_ln']
    dim = x.shape[1]
    te9 = p['type_emb'] @ w1[2 * dim + 1:]
    nodes = _call_rows(
        _enc_node_body, n_nodes, bn,
        [x, v, rho.reshape(-1, 1), particle_type.reshape(-1, 1).astype(jnp.int32)],
        [w1[:dim], w1[dim:2 * dim], w1[2 * dim:2 * dim + 1], te9, r2(b1),
         w2, r2(b2), w3, r2(b3), r2(lng), r2(lnb)],
        jax.ShapeDtypeStruct((n_nodes, w3.shape[1]), f32))

    enc_e = p['enc_edge']
    enc_e_ln = p['enc_edge_ln']
    edges = None

    latent = nodes.shape[1]
    nsteps = len(p['proc'])
    for step, pp in enumerate(p['proc']):
        (w1, b1), (w2, b2), (w3, b3) = pp['edge_mlp']
        lng, lnb = pp['edge_ln']
        w1e = w1[:latent]
        w1s = w1[latent:2 * latent]
        w1d = w1[2 * latent:]
        ts, td = _call_rows(
            _tables_body, n_nodes, bn, [nodes], [w1s, w1d, r2(b1)],
            [jax.ShapeDtypeStruct((n_nodes, 2 * latent), f32),
             jax.ShapeDtypeStruct((n_nodes, 2 * latent), f32)])
        g = _sc_gather_add(ts, td, src, dst)
        if step == 0:
            (ew1, eb1), (ew2, eb2), (ew3, eb3) = enc_e
            elng, elnb = enc_e_ln
            e_upd_m, edges = _call_rows(
                _edge_first_body, n_edges, be, [edge_features, g, dst2],
                [ew1, r2(eb1), ew2, r2(eb2), ew3, r2(eb3), r2(elng), r2(elnb),
                 w1e, w2, r2(b2), w3, r2(b3), r2(lng), r2(lnb)],
                [jax.ShapeDtypeStruct((n_edges, 2 * latent), f32),
                 jax.ShapeDtypeStruct((n_edges, latent), f32)])
        elif step < nsteps - 1:
            e_upd_m, edges = _call_rows(
                _edge_step_body, n_edges, be, [edges, g, dst2],
                [w1e, w2, r2(b2), w3, r2(b3), r2(lng), r2(lnb)],
                [jax.ShapeDtypeStruct((n_edges, 2 * latent), f32),
                 jax.ShapeDtypeStruct((n_edges, latent), f32)])
        else:
            e_upd_m = _call_rows(
                _edge_step_last_body, n_edges, be, [edges, g, dst2],
                [w1e, w2, r2(b2), w3, r2(b3), r2(lng), r2(lnb)],
                jax.ShapeDtypeStruct((n_edges, 2 * latent), f32))
        agg = _sc_segment_sum(e_upd_m, dst, n_nodes).reshape(n_nodes, latent)
        (w1, b1), (w2, b2), (w3, b3) = pp['node_mlp']
        lng, lnb = pp['node_ln']
        nodes = _call_rows(
            _node_step_body, n_nodes, bn, [nodes, agg],
            [w1[:latent], w1[latent:], r2(b1), w2, r2(b2), w3, r2(b3),
             r2(lng), r2(lnb)],
            jax.ShapeDtypeStruct((n_nodes, latent), f32))

    (w1, b1), (w2, b2), (w3, b3) = p['dec']
    return _call_rows(
        _decode_body, n_nodes, bn, [nodes],
        [w1, r2(b1), w2, r2(b2), w3, r2(b3)],
        jax.ShapeDtypeStruct((n_nodes, w3.shape[1]), f32))
